# baseline (speedup 1.0000x reference)
import functools

import jax
import jax.numpy as jnp
from jax.experimental import pallas as pl
from jax.experimental.pallas import tpu as pltpu
from jax.experimental.pallas import tpu_sc as plsc

B, N, D, K = 16, 576, 256, 8192
BN = B * N
TM = 1024
T = BN // TM
KBLK = 4096
KC = K // KBLK

_NC, _NS = 2, 16
_NW = _NC * _NS
_BPW = BN // _NW


def _argmin_body(w_ref, pw_ref, pb_ref, z_ref, emb_ref, en_ref, idx_ref):
    @pl.when(pl.program_id(0) == 0)
    def _():
        e_full = jax.lax.dot_general(
            w_ref[...], pw_ref[...], (((1,), (1,)), ((), ())),
            preferred_element_type=jnp.float32) + pb_ref[...]
        emb_ref[...] = e_full
        en_ref[...] = jnp.sum(e_full * e_full, axis=1)[None, :]

    z_t = z_ref[...]
    znorm = jnp.sum(z_t * z_t, axis=1, keepdims=True)
    z2 = z_t * jnp.float32(-2.0)
    colf = jax.lax.broadcasted_iota(
        jnp.int32, (TM, KBLK), 1).astype(jnp.float32)

    run_min = jnp.full((TM,), jnp.inf, dtype=jnp.float32)
    run_idx = jnp.zeros((TM,), dtype=jnp.float32)
    for c in range(KC):
        e = emb_ref[c * KBLK:(c + 1) * KBLK, :]
        enorm = en_ref[0:1, c * KBLK:(c + 1) * KBLK]
        s = jax.lax.dot_general(z2, e, (((1,), (1,)), ((), ())),
                                preferred_element_type=jnp.float32)
        d = (znorm + enorm) + s
        m = jnp.min(d, axis=1)
        ic = jnp.min(jnp.where(d == m[:, None], colf, jnp.float32(1e9)),
                     axis=1) + jnp.float32(c * KBLK)
        better = m < run_min
        run_idx = jnp.where(better, ic, run_idx)
        run_min = jnp.minimum(run_min, m)
    idx_ref[0, 0, :] = run_idx.astype(jnp.int32)


_argmin_call = pl.pallas_call(
    _argmin_body,
    grid=(T,),
    in_specs=[
        pl.BlockSpec((K, D), lambda t: (0, 0)),
        pl.BlockSpec((D, D), lambda t: (0, 0)),
        pl.BlockSpec((1, D), lambda t: (0, 0)),
        pl.BlockSpec((TM, D), lambda t: (t, 0)),
    ],
    out_specs=[
        pl.BlockSpec((K, D), lambda t: (0, 0)),
        pl.BlockSpec((1, K), lambda t: (0, 0)),
        pl.BlockSpec((1, 1, TM), lambda t: (t, 0, 0)),
    ],
    out_shape=[
        jax.ShapeDtypeStruct((K, D), jnp.float32),
        jax.ShapeDtypeStruct((1, K), jnp.float32),
        jax.ShapeDtypeStruct((T, 1, TM), jnp.int32),
    ],
)


@functools.cache
def _make_sc_gather():
    mesh = plsc.VectorSubcoreMesh(core_axis_name="c", subcore_axis_name="s")

    @functools.partial(
        pl.kernel,
        mesh=mesh,
        out_type=jax.ShapeDtypeStruct((BN, D), jnp.float32),
        scratch_types=[
            pltpu.VMEM((_BPW,), jnp.int32),
            pltpu.VMEM((_BPW, D), jnp.float32),
            pltpu.SemaphoreType.DMA,
        ],
    )
    def _sc_gather(table_hbm, idx_hbm, out_hbm, idx_v, rows_v, sem):
        wid = jax.lax.axis_index("s") * _NC + jax.lax.axis_index("c")
        base = wid * _BPW
        pltpu.sync_copy(idx_hbm.at[pl.ds(base, _BPW)], idx_v)
        pltpu.async_copy(table_hbm.at[idx_v], rows_v, sem).wait()
        pltpu.sync_copy(rows_v, out_hbm.at[pl.ds(base, _BPW)])

    return _sc_gather


def kernel(z, emb_weight, proj_w, proj_b):
    z_flat = z.reshape(BN, D)
    emb, _, idx3 = _argmin_call(emb_weight, proj_w, proj_b.reshape(1, D), z_flat)
    q_idx = idx3.reshape(BN)
    rows = _make_sc_gather()(emb, q_idx)
    quantized = rows.reshape(z.shape)
    return (z, emb, quantized, q_idx.reshape(B, N))


# speedup vs baseline: 1.0134x; 1.0134x over previous
import functools

import jax
import jax.numpy as jnp
from jax.experimental import pallas as pl
from jax.experimental.pallas import tpu as pltpu
from jax.experimental.pallas import tpu_sc as plsc

B, N, D, K = 16, 576, 256, 8192
BN = B * N
TM = 1152
T = BN // TM
KBLK = 2048
KC = K // KBLK

_NC, _NS = 2, 16
_NW = _NC * _NS
_BPW = BN // _NW


def _argmin_body(w_ref, pw_ref, pb_ref, z_ref, emb_ref, en_ref, idx_ref):
    @pl.when(pl.program_id(0) == 0)
    def _():
        e_full = jax.lax.dot_general(
            w_ref[...], pw_ref[...], (((1,), (1,)), ((), ())),
            preferred_element_type=jnp.float32) + pb_ref[...]
        emb_ref[...] = e_full
        en_ref[...] = jnp.sum(e_full * e_full, axis=1)[None, :]

    z_t = z_ref[...]
    znorm = jnp.sum(z_t * z_t, axis=1, keepdims=True)
    z2 = z_t * jnp.float32(-2.0)
    colf = jax.lax.broadcasted_iota(
        jnp.int32, (TM, KBLK), 1).astype(jnp.float32)

    run_min = jnp.full((TM,), jnp.inf, dtype=jnp.float32)
    run_idx = jnp.zeros((TM,), dtype=jnp.float32)
    for c in range(KC):
        e = emb_ref[c * KBLK:(c + 1) * KBLK, :]
        enorm = en_ref[0:1, c * KBLK:(c + 1) * KBLK]
        s = jax.lax.dot_general(z2, e, (((1,), (1,)), ((), ())),
                                preferred_element_type=jnp.float32)
        d = (znorm + enorm) + s
        m = jnp.min(d, axis=1)
        ic = jnp.min(jnp.where(d == m[:, None], colf, jnp.float32(1e9)),
                     axis=1) + jnp.float32(c * KBLK)
        better = m < run_min
        run_idx = jnp.where(better, ic, run_idx)
        run_min = jnp.minimum(run_min, m)
    idx_ref[0, 0, :] = run_idx.astype(jnp.int32)


_argmin_call = pl.pallas_call(
    _argmin_body,
    grid=(T,),
    in_specs=[
        pl.BlockSpec((K, D), lambda t: (0, 0)),
        pl.BlockSpec((D, D), lambda t: (0, 0)),
        pl.BlockSpec((1, D), lambda t: (0, 0)),
        pl.BlockSpec((TM, D), lambda t: (t, 0)),
    ],
    out_specs=[
        pl.BlockSpec((K, D), lambda t: (0, 0)),
        pl.BlockSpec((1, K), lambda t: (0, 0)),
        pl.BlockSpec((1, 1, TM), lambda t: (t, 0, 0)),
    ],
    out_shape=[
        jax.ShapeDtypeStruct((K, D), jnp.float32),
        jax.ShapeDtypeStruct((1, K), jnp.float32),
        jax.ShapeDtypeStruct((T, 1, TM), jnp.int32),
    ],
)


@functools.cache
def _make_sc_gather():
    mesh = plsc.VectorSubcoreMesh(core_axis_name="c", subcore_axis_name="s")

    @functools.partial(
        pl.kernel,
        mesh=mesh,
        out_type=jax.ShapeDtypeStruct((BN, D), jnp.float32),
        scratch_types=[
            pltpu.VMEM((_BPW,), jnp.int32),
            pltpu.VMEM((_BPW, D), jnp.float32),
            pltpu.SemaphoreType.DMA,
        ],
    )
    def _sc_gather(table_hbm, idx_hbm, out_hbm, idx_v, rows_v, sem):
        wid = jax.lax.axis_index("s") * _NC + jax.lax.axis_index("c")
        base = wid * _BPW
        pltpu.sync_copy(idx_hbm.at[pl.ds(base, _BPW)], idx_v)
        pltpu.async_copy(table_hbm.at[idx_v], rows_v, sem).wait()
        pltpu.sync_copy(rows_v, out_hbm.at[pl.ds(base, _BPW)])

    return _sc_gather


def kernel(z, emb_weight, proj_w, proj_b):
    z_flat = z.reshape(BN, D)
    emb, _, idx3 = _argmin_call(emb_weight, proj_w, proj_b.reshape(1, D), z_flat)
    q_idx = idx3.reshape(BN)
    rows = _make_sc_gather()(emb, q_idx)
    quantized = rows.reshape(z.shape)
    return (z, emb, quantized, q_idx.reshape(B, N))


# speedup vs baseline: 1.0275x; 1.0139x over previous
import functools

import jax
import jax.numpy as jnp
from jax.experimental import pallas as pl
from jax.experimental.pallas import tpu as pltpu
from jax.experimental.pallas import tpu_sc as plsc

B, N, D, K = 16, 576, 256, 8192
BN = B * N
TM = 1152
T = BN // TM
KBLK = 1024
KC = K // KBLK

_NC, _NS = 2, 16
_NW = _NC * _NS
_BPW = BN // _NW


def _argmin_body(w_ref, pw_ref, pb_ref, z_ref, emb_ref, en_ref, idx_ref):
    @pl.when(pl.program_id(0) == 0)
    def _():
        e_full = jax.lax.dot_general(
            w_ref[...], pw_ref[...], (((1,), (1,)), ((), ())),
            preferred_element_type=jnp.float32) + pb_ref[...]
        emb_ref[...] = e_full
        en_ref[...] = jnp.sum(e_full * e_full, axis=1)[None, :]

    z_t = z_ref[...]
    znorm = jnp.sum(z_t * z_t, axis=1, keepdims=True)
    z2 = z_t * jnp.float32(-2.0)
    colf = jax.lax.broadcasted_iota(
        jnp.int32, (TM, KBLK), 1).astype(jnp.float32)

    run_min = jnp.full((TM,), jnp.inf, dtype=jnp.float32)
    run_idx = jnp.zeros((TM,), dtype=jnp.float32)
    for c in range(KC):
        e = emb_ref[c * KBLK:(c + 1) * KBLK, :]
        enorm = en_ref[0:1, c * KBLK:(c + 1) * KBLK]
        s = jax.lax.dot_general(z2, e, (((1,), (1,)), ((), ())),
                                preferred_element_type=jnp.float32)
        d = (znorm + enorm) + s
        m = jnp.min(d, axis=1)
        ic = jnp.min(jnp.where(d == m[:, None], colf, jnp.float32(1e9)),
                     axis=1) + jnp.float32(c * KBLK)
        better = m < run_min
        run_idx = jnp.where(better, ic, run_idx)
        run_min = jnp.minimum(run_min, m)
    idx_ref[0, 0, :] = run_idx.astype(jnp.int32)


_argmin_call = pl.pallas_call(
    _argmin_body,
    grid=(T,),
    in_specs=[
        pl.BlockSpec((K, D), lambda t: (0, 0)),
        pl.BlockSpec((D, D), lambda t: (0, 0)),
        pl.BlockSpec((1, D), lambda t: (0, 0)),
        pl.BlockSpec((TM, D), lambda t: (t, 0)),
    ],
    out_specs=[
        pl.BlockSpec((K, D), lambda t: (0, 0)),
        pl.BlockSpec((1, K), lambda t: (0, 0)),
        pl.BlockSpec((1, 1, TM), lambda t: (t, 0, 0)),
    ],
    out_shape=[
        jax.ShapeDtypeStruct((K, D), jnp.float32),
        jax.ShapeDtypeStruct((1, K), jnp.float32),
        jax.ShapeDtypeStruct((T, 1, TM), jnp.int32),
    ],
)


@functools.cache
def _make_sc_gather():
    mesh = plsc.VectorSubcoreMesh(core_axis_name="c", subcore_axis_name="s")

    @functools.partial(
        pl.kernel,
        mesh=mesh,
        out_type=jax.ShapeDtypeStruct((BN, D), jnp.float32),
        scratch_types=[
            pltpu.VMEM((_BPW,), jnp.int32),
            pltpu.VMEM((_BPW, D), jnp.float32),
            pltpu.SemaphoreType.DMA,
        ],
    )
    def _sc_gather(table_hbm, idx_hbm, out_hbm, idx_v, rows_v, sem):
        wid = jax.lax.axis_index("s") * _NC + jax.lax.axis_index("c")
        base = wid * _BPW
        pltpu.sync_copy(idx_hbm.at[pl.ds(base, _BPW)], idx_v)
        pltpu.async_copy(table_hbm.at[idx_v], rows_v, sem).wait()
        pltpu.sync_copy(rows_v, out_hbm.at[pl.ds(base, _BPW)])

    return _sc_gather


def kernel(z, emb_weight, proj_w, proj_b):
    z_flat = z.reshape(BN, D)
    emb, _, idx3 = _argmin_call(emb_weight, proj_w, proj_b.reshape(1, D), z_flat)
    q_idx = idx3.reshape(BN)
    rows = _make_sc_gather()(emb, q_idx)
    quantized = rows.reshape(z.shape)
    return (z, emb, quantized, q_idx.reshape(B, N))


# speedup vs baseline: 1.0287x; 1.0012x over previous
import functools

import jax
import jax.numpy as jnp
from jax.experimental import pallas as pl
from jax.experimental.pallas import tpu as pltpu
from jax.experimental.pallas import tpu_sc as plsc

B, N, D, K = 16, 576, 256, 8192
BN = B * N
TM = 1152
T = BN // TM
KBLK = 512
KC = K // KBLK

_NC, _NS = 2, 16
_NW = _NC * _NS
_BPW = BN // _NW


def _argmin_body(w_ref, pw_ref, pb_ref, z_ref, emb_ref, en_ref, idx_ref):
    @pl.when(pl.program_id(0) == 0)
    def _():
        e_full = jax.lax.dot_general(
            w_ref[...], pw_ref[...], (((1,), (1,)), ((), ())),
            preferred_element_type=jnp.float32) + pb_ref[...]
        emb_ref[...] = e_full
        en_ref[...] = jnp.sum(e_full * e_full, axis=1)[None, :]

    z_t = z_ref[...]
    znorm = jnp.sum(z_t * z_t, axis=1, keepdims=True)
    z2 = z_t * jnp.float32(-2.0)
    colf = jax.lax.broadcasted_iota(
        jnp.int32, (TM, KBLK), 1).astype(jnp.float32)

    run_min = jnp.full((TM,), jnp.inf, dtype=jnp.float32)
    run_idx = jnp.zeros((TM,), dtype=jnp.float32)
    for c in range(KC):
        e = emb_ref[c * KBLK:(c + 1) * KBLK, :]
        enorm = en_ref[0:1, c * KBLK:(c + 1) * KBLK]
        s = jax.lax.dot_general(z2, e, (((1,), (1,)), ((), ())),
                                preferred_element_type=jnp.float32)
        d = (znorm + enorm) + s
        m = jnp.min(d, axis=1)
        ic = jnp.min(jnp.where(d == m[:, None], colf, jnp.float32(1e9)),
                     axis=1) + jnp.float32(c * KBLK)
        better = m < run_min
        run_idx = jnp.where(better, ic, run_idx)
        run_min = jnp.minimum(run_min, m)
    idx_ref[0, 0, :] = run_idx.astype(jnp.int32)


_argmin_call = pl.pallas_call(
    _argmin_body,
    grid=(T,),
    in_specs=[
        pl.BlockSpec((K, D), lambda t: (0, 0)),
        pl.BlockSpec((D, D), lambda t: (0, 0)),
        pl.BlockSpec((1, D), lambda t: (0, 0)),
        pl.BlockSpec((TM, D), lambda t: (t, 0)),
    ],
    out_specs=[
        pl.BlockSpec((K, D), lambda t: (0, 0)),
        pl.BlockSpec((1, K), lambda t: (0, 0)),
        pl.BlockSpec((1, 1, TM), lambda t: (t, 0, 0)),
    ],
    out_shape=[
        jax.ShapeDtypeStruct((K, D), jnp.float32),
        jax.ShapeDtypeStruct((1, K), jnp.float32),
        jax.ShapeDtypeStruct((T, 1, TM), jnp.int32),
    ],
)


@functools.cache
def _make_sc_gather():
    mesh = plsc.VectorSubcoreMesh(core_axis_name="c", subcore_axis_name="s")

    @functools.partial(
        pl.kernel,
        mesh=mesh,
        out_type=jax.ShapeDtypeStruct((BN, D), jnp.float32),
        scratch_types=[
            pltpu.VMEM((_BPW,), jnp.int32),
            pltpu.VMEM((_BPW, D), jnp.float32),
            pltpu.SemaphoreType.DMA,
        ],
    )
    def _sc_gather(table_hbm, idx_hbm, out_hbm, idx_v, rows_v, sem):
        wid = jax.lax.axis_index("s") * _NC + jax.lax.axis_index("c")
        base = wid * _BPW
        pltpu.sync_copy(idx_hbm.at[pl.ds(base, _BPW)], idx_v)
        pltpu.async_copy(table_hbm.at[idx_v], rows_v, sem).wait()
        pltpu.sync_copy(rows_v, out_hbm.at[pl.ds(base, _BPW)])

    return _sc_gather


def kernel(z, emb_weight, proj_w, proj_b):
    z_flat = z.reshape(BN, D)
    emb, _, idx3 = _argmin_call(emb_weight, proj_w, proj_b.reshape(1, D), z_flat)
    q_idx = idx3.reshape(BN)
    rows = _make_sc_gather()(emb, q_idx)
    quantized = rows.reshape(z.shape)
    return (z, emb, quantized, q_idx.reshape(B, N))
